# Initial kernel scaffold; baseline (speedup 1.0000x reference)
import jax
import jax.numpy as jnp
from jax.experimental import pallas as pl


def kernel(h, edge_index, W1, b1, W2, b2, W3, b3):
    raise NotImplementedError("write your pallas kernel here")


# speedup vs baseline: 3.5483x; 3.5483x over previous
import functools

import jax
import jax.numpy as jnp
from jax import lax
from jax.experimental import pallas as pl
from jax.experimental.pallas import tpu as pltpu
from jax.experimental.pallas import tpu_sc as plsc

_NC = 2
_NS = 16
_NW = _NC * _NS
_K = 128


def _sc_aggregate(x, src, dst, n_pad):
    n, d = x.shape
    epad = src.shape[0]
    nchunks = epad // (_NW * _K)
    assert nchunks * _NW * _K == epad
    rows_per_sub = n_pad // _NS

    mesh = plsc.VectorSubcoreMesh(core_axis_name="c", subcore_axis_name="s")

    @functools.partial(
        pl.kernel,
        mesh=mesh,
        out_type=(
            jax.ShapeDtypeStruct((n_pad, d), jnp.float32),
            jax.ShapeDtypeStruct((n_pad, d), jnp.float32),
        ),
        scratch_types=[
            pltpu.VMEM((_K,), jnp.int32),
            pltpu.VMEM((_K,), jnp.int32),
            pltpu.VMEM((_K, d), jnp.float32),
            pltpu.VMEM_SHARED((n_pad, d), jnp.float32),
            pltpu.SemaphoreType.DMA,
        ],
    )
    def agg_kernel(x_hbm, src_hbm, dst_hbm, agg0_hbm, agg1_hbm,
                   src_v, dst_v, rows_v, agg_s, sem):
        cid = lax.axis_index("c")
        sid = lax.axis_index("s")
        wid = sid * _NC + cid

        def zbody(i, c):
            for j in range(d // 16):
                rows_v[i, pl.ds(j * 16, 16)] = jnp.zeros((16,), jnp.float32)
            return c
        lax.fori_loop(0, _K, zbody, 0)
        for j in range(rows_per_sub // _K):
            pltpu.sync_copy(
                rows_v, agg_s.at[pl.ds(sid * rows_per_sub + j * _K, _K)])
        plsc.subcore_barrier()

        def body(ci, c):
            base = pl.multiple_of((wid * nchunks + ci) * _K, _K)
            pltpu.sync_copy(src_hbm.at[pl.ds(base, _K)], src_v)
            pltpu.sync_copy(dst_hbm.at[pl.ds(base, _K)], dst_v)
            pltpu.async_copy(x_hbm.at[src_v], rows_v, sem).wait()
            pltpu.sync_copy(rows_v, agg_s.at[dst_v], add=True)
            return c
        lax.fori_loop(0, nchunks, body, 0)
        plsc.subcore_barrier()

        wbase = pl.multiple_of(sid * rows_per_sub, _K)

        @pl.when(cid == 0)
        def _():
            pltpu.sync_copy(agg_s.at[pl.ds(wbase, rows_per_sub)],
                            agg0_hbm.at[pl.ds(wbase, rows_per_sub)])

        @pl.when(cid == 1)
        def _():
            pltpu.sync_copy(agg_s.at[pl.ds(wbase, rows_per_sub)],
                            agg1_hbm.at[pl.ds(wbase, rows_per_sub)])

    return agg_kernel(x, src, dst)


def _tc_update(x, a0, a1, w, b2d):
    n, d = x.shape

    def body(x_ref, a0_ref, a1_ref, w_ref, b_ref, xo_ref, s_ref):
        m = x_ref[...] + a0_ref[:n] + a1_ref[:n]
        y = jnp.dot(m, w_ref[...], preferred_element_type=jnp.float32)
        y = jnp.maximum(y + b_ref[...], 0.0)
        xo_ref[...] = y
        s_ref[...] = jnp.sum(y, axis=0, keepdims=True)

    return pl.pallas_call(
        body,
        out_shape=(
            jax.ShapeDtypeStruct((n, d), jnp.float32),
            jax.ShapeDtypeStruct((1, d), jnp.float32),
        ),
    )(x, a0, a1, w, b2d)


def kernel(h, edge_index, W1, b1, W2, b2, W3, b3):
    n, d = h.shape
    e = edge_index.shape[1]
    quantum = _NW * _K
    epad = ((e + quantum - 1) // quantum) * quantum
    n_pad = ((n + 1 + _NS * _K - 1) // (_NS * _K)) * (_NS * _K)

    src = jnp.concatenate(
        [edge_index[0], jnp.zeros((epad - e,), jnp.int32)])
    dst = jnp.concatenate(
        [edge_index[1], jnp.full((epad - e,), n, jnp.int32)])

    x = h
    sums = []
    for (w, b) in ((W1, b1), (W2, b2), (W3, b3)):
        a0, a1 = _sc_aggregate(x, src, dst, n_pad)
        x, s = _tc_update(x, a0, a1, w, b.reshape(1, d))
        sums.append(s[0])
    return jnp.concatenate(sums)
